# baseline (speedup 1.0000x reference)
import functools

import jax
import jax.numpy as jnp
from jax import lax
from jax.experimental import pallas as pl
from jax.experimental.pallas import tpu as pltpu
from jax.experimental.pallas import tpu_sc as plsc

N = 16384
K = 1024
DIM = 256
BN = 1024
HALF = N // 2
HGRID = HALF // BN


def _dist_argmin(x, ct_ref, b2_ref):
    t = x * x
    u = t[:, 0:128] + t[:, 128:256]
    ut = jnp.transpose(u)
    acc = ut[0:8, :]
    for v in range(1, 16):
        acc = acc + ut[8 * v:8 * v + 8, :]
    h4 = acc[0:4, :] + acc[4:8, :]
    h2 = h4[0:2, :] + h4[2:4, :]
    a2 = jnp.transpose(h2[0:1, :] + h2[1:2, :])
    s = lax.dot_general(
        x.astype(jnp.bfloat16), ct_ref[...],
        dimension_numbers=(((1,), (0,)), ((), ())),
        preferred_element_type=jnp.float32,
    )
    b2 = b2_ref[0:1, :]
    d2 = (a2 + b2) - 2.0 * s
    dc = jnp.clip(d2, 0.0, None)
    dist = dc * lax.rsqrt(dc)
    m = dist[:, 0:128]
    cid = jnp.zeros((BN, 128), jnp.int32)
    for j in range(1, K // 128):
        c = dist[:, j * 128:(j + 1) * 128]
        lt = c < m
        m = jnp.where(lt, c, m)
        cid = jnp.where(lt, j, cid)
    mm = jnp.min(m, axis=1, keepdims=True)
    lane = lax.broadcasted_iota(jnp.int32, (BN, 128), 1)
    cand = jnp.where(m == mm, cid * 128 + lane, jnp.int32(K))
    return jnp.reshape(jnp.min(cand, axis=1, keepdims=True), (BN // 128, 128))


def _round0_body(x_ref, ct_ref, b2_ref, idx_ref):
    idx_ref[...] = _dist_argmin(x_ref[...], ct_ref, b2_ref)


def _round_body(x_ref, g_ref, ct_ref, b2_ref, idx_ref, xout_ref):
    x = x_ref[...] - g_ref[...]
    xout_ref[...] = x
    idx_ref[...] = _dist_argmin(x, ct_ref, b2_ref)


def _sub_body(x_ref, g_ref, out_ref):
    out_ref[...] = x_ref[...] - g_ref[...]


_CTSPEC = pl.BlockSpec((DIM, K), lambda i: (0, 0))
_B2SPEC = pl.BlockSpec((8, K), lambda i: (0, 0))
_IDXSPEC = pl.BlockSpec((BN // 128, 128), lambda i: (i, 0))
_HSPEC = pl.BlockSpec((BN, DIM), lambda i: (i, 0))

_IDX_SHAPE = jax.ShapeDtypeStruct((HALF // 128, 128), jnp.int32)
_X_SHAPE = jax.ShapeDtypeStruct((HALF, DIM), jnp.float32)


def _xfull_spec(off):
    return pl.BlockSpec((BN, DIM), lambda i, o=off: (i + o, 0))


def _make_round0(off):
    return pl.pallas_call(
        _round0_body,
        grid=(HGRID,),
        in_specs=[_xfull_spec(off), _CTSPEC, _B2SPEC],
        out_specs=_IDXSPEC,
        out_shape=_IDX_SHAPE,
    )


def _make_round1(off):
    return pl.pallas_call(
        _round_body,
        grid=(HGRID,),
        in_specs=[_xfull_spec(off), _HSPEC, _CTSPEC, _B2SPEC],
        out_specs=[_IDXSPEC, _HSPEC],
        out_shape=[_IDX_SHAPE, _X_SHAPE],
    )


_round0_a = _make_round0(0)
_round0_b = _make_round0(HGRID)
_round1_a = _make_round1(0)
_round1_b = _make_round1(HGRID)

_round_h = pl.pallas_call(
    _round_body,
    grid=(HGRID,),
    in_specs=[_HSPEC, _HSPEC, _CTSPEC, _B2SPEC],
    out_specs=[_IDXSPEC, _HSPEC],
    out_shape=[_IDX_SHAPE, _X_SHAPE],
)

_sub_h = pl.pallas_call(
    _sub_body,
    grid=(HGRID,),
    in_specs=[_HSPEC, _HSPEC],
    out_specs=_HSPEC,
    out_shape=_X_SHAPE,
)


def _make_gather(rows):
    info = plsc.get_sparse_core_info()
    nc, ns = info.num_cores, info.num_subcores
    nw = nc * ns
    b_per_w = rows // nw
    mesh = plsc.VectorSubcoreMesh(core_axis_name="c", subcore_axis_name="s")

    @functools.partial(
        pl.kernel,
        out_type=jax.ShapeDtypeStruct((rows, DIM), jnp.float32),
        mesh=mesh,
        scratch_types=[
            pltpu.VMEM((b_per_w,), jnp.int32),
            pltpu.VMEM((b_per_w, DIM), jnp.float32),
            pltpu.SemaphoreType.DMA,
        ],
    )
    def gather(table_hbm, idx_hbm, out_hbm, idx_v, rows_v, sem):
        wid = lax.axis_index("s") * nc + lax.axis_index("c")
        base = wid * b_per_w
        pltpu.sync_copy(idx_hbm.at[pl.ds(base, b_per_w)], idx_v)
        half = b_per_w // 2
        g0 = pltpu.async_copy(table_hbm.at[idx_v.at[pl.ds(0, half)]],
                              rows_v.at[pl.ds(0, half)], sem)
        g1 = pltpu.async_copy(table_hbm.at[idx_v.at[pl.ds(half, half)]],
                              rows_v.at[pl.ds(half, half)], sem)
        g0.wait()
        g1.wait()
        pltpu.sync_copy(rows_v, out_hbm.at[pl.ds(base, b_per_w)])

    return gather


_gather = _make_gather(HALF)


def _chain(full_x, round0, round1, books, cts, b2s):
    idx0 = round0(full_x, cts[0], b2s[0])
    g0 = _gather(books[0], jnp.reshape(idx0, (HALF,)))
    idx1, x1 = round1(full_x, g0, cts[1], b2s[1])
    g1 = _gather(books[1], jnp.reshape(idx1, (HALF,)))
    idx2, x2 = _round_h(x1, g1, cts[2], b2s[2])
    g2 = _gather(books[2], jnp.reshape(idx2, (HALF,)))
    idx3, x3 = _round_h(x2, g2, cts[3], b2s[3])
    g3 = _gather(books[3], jnp.reshape(idx3, (HALF,)))
    res = _sub_h(x3, g3)
    out = jnp.stack([jnp.reshape(i, (HALF,)) for i in (idx0, idx1, idx2, idx3)],
                    axis=-1)
    return out, res


def kernel(input, codebook_0, codebook_1, codebook_2, codebook_3):
    books = [codebook_0, codebook_1, codebook_2, codebook_3]
    cts = [jnp.transpose(c).astype(jnp.bfloat16) for c in books]
    b2s = [
        jnp.broadcast_to(jnp.sum(c * c, axis=-1)[None, :], (8, K)) for c in books
    ]

    out_a, res_a = _chain(input, _round0_a, _round1_a, books, cts, b2s)
    out_b, res_b = _chain(input, _round0_b, _round1_b, books, cts, b2s)

    output = jnp.concatenate([out_a, out_b], axis=0)
    res = jnp.concatenate([res_a, res_b], axis=0)
    return (output, res)


# speedup vs baseline: 1.0318x; 1.0318x over previous
import functools

import jax
import jax.numpy as jnp
from jax import lax
from jax.experimental import pallas as pl
from jax.experimental.pallas import tpu as pltpu
from jax.experimental.pallas import tpu_sc as plsc

N = 16384
K = 1024
DIM = 256
BN = 2048
HALF = N // 2
HGRID = HALF // BN


def _dist_argmin(x, ct_ref, b2_ref):
    t = x * x
    u = t[:, 0:128] + t[:, 128:256]
    ut = jnp.transpose(u)
    acc = ut[0:8, :]
    for v in range(1, 16):
        acc = acc + ut[8 * v:8 * v + 8, :]
    h4 = acc[0:4, :] + acc[4:8, :]
    h2 = h4[0:2, :] + h4[2:4, :]
    a2 = jnp.transpose(h2[0:1, :] + h2[1:2, :])
    s = lax.dot_general(
        x.astype(jnp.bfloat16), ct_ref[...],
        dimension_numbers=(((1,), (0,)), ((), ())),
        preferred_element_type=jnp.float32,
    )
    b2 = b2_ref[0:1, :]
    d2 = (a2 + b2) - 2.0 * s
    dc = jnp.clip(d2, 0.0, None)
    dist = dc * lax.rsqrt(dc)
    m = dist[:, 0:128]
    cid = jnp.zeros((BN, 128), jnp.int32)
    for j in range(1, K // 128):
        c = dist[:, j * 128:(j + 1) * 128]
        lt = c < m
        m = jnp.where(lt, c, m)
        cid = jnp.where(lt, j, cid)
    mm = jnp.min(m, axis=1, keepdims=True)
    lane = lax.broadcasted_iota(jnp.int32, (BN, 128), 1)
    cand = jnp.where(m == mm, cid * 128 + lane, jnp.int32(K))
    return jnp.reshape(jnp.min(cand, axis=1, keepdims=True), (BN // 128, 128))


def _round0_body(x_ref, ct_ref, b2_ref, idx_ref):
    idx_ref[...] = _dist_argmin(x_ref[...], ct_ref, b2_ref)


def _round_body(x_ref, g_ref, ct_ref, b2_ref, idx_ref, xout_ref):
    x = x_ref[...] - g_ref[...]
    xout_ref[...] = x
    idx_ref[...] = _dist_argmin(x, ct_ref, b2_ref)


def _sub_body(x_ref, g_ref, out_ref):
    out_ref[...] = x_ref[...] - g_ref[...]


_CTSPEC = pl.BlockSpec((DIM, K), lambda i: (0, 0))
_B2SPEC = pl.BlockSpec((8, K), lambda i: (0, 0))
_IDXSPEC = pl.BlockSpec((BN // 128, 128), lambda i: (i, 0))
_HSPEC = pl.BlockSpec((BN, DIM), lambda i: (i, 0))

_IDX_SHAPE = jax.ShapeDtypeStruct((HALF // 128, 128), jnp.int32)
_X_SHAPE = jax.ShapeDtypeStruct((HALF, DIM), jnp.float32)


def _xfull_spec(off):
    return pl.BlockSpec((BN, DIM), lambda i, o=off: (i + o, 0))


def _make_round0(off):
    return pl.pallas_call(
        _round0_body,
        grid=(HGRID,),
        in_specs=[_xfull_spec(off), _CTSPEC, _B2SPEC],
        out_specs=_IDXSPEC,
        out_shape=_IDX_SHAPE,
    )


def _make_round1(off):
    return pl.pallas_call(
        _round_body,
        grid=(HGRID,),
        in_specs=[_xfull_spec(off), _HSPEC, _CTSPEC, _B2SPEC],
        out_specs=[_IDXSPEC, _HSPEC],
        out_shape=[_IDX_SHAPE, _X_SHAPE],
    )


_round0_a = _make_round0(0)
_round0_b = _make_round0(HGRID)
_round1_a = _make_round1(0)
_round1_b = _make_round1(HGRID)

_round_h = pl.pallas_call(
    _round_body,
    grid=(HGRID,),
    in_specs=[_HSPEC, _HSPEC, _CTSPEC, _B2SPEC],
    out_specs=[_IDXSPEC, _HSPEC],
    out_shape=[_IDX_SHAPE, _X_SHAPE],
)

_sub_h = pl.pallas_call(
    _sub_body,
    grid=(HGRID,),
    in_specs=[_HSPEC, _HSPEC],
    out_specs=_HSPEC,
    out_shape=_X_SHAPE,
)


def _make_gather(rows):
    info = plsc.get_sparse_core_info()
    nc, ns = info.num_cores, info.num_subcores
    nw = nc * ns
    b_per_w = rows // nw
    mesh = plsc.VectorSubcoreMesh(core_axis_name="c", subcore_axis_name="s")

    @functools.partial(
        pl.kernel,
        out_type=jax.ShapeDtypeStruct((rows, DIM), jnp.float32),
        mesh=mesh,
        scratch_types=[
            pltpu.VMEM((b_per_w,), jnp.int32),
            pltpu.VMEM((b_per_w, DIM), jnp.float32),
            pltpu.SemaphoreType.DMA,
        ],
    )
    def gather(table_hbm, idx_hbm, out_hbm, idx_v, rows_v, sem):
        wid = lax.axis_index("s") * nc + lax.axis_index("c")
        base = wid * b_per_w
        pltpu.sync_copy(idx_hbm.at[pl.ds(base, b_per_w)], idx_v)
        half = b_per_w // 2
        g0 = pltpu.async_copy(table_hbm.at[idx_v.at[pl.ds(0, half)]],
                              rows_v.at[pl.ds(0, half)], sem)
        g1 = pltpu.async_copy(table_hbm.at[idx_v.at[pl.ds(half, half)]],
                              rows_v.at[pl.ds(half, half)], sem)
        g0.wait()
        g1.wait()
        pltpu.sync_copy(rows_v, out_hbm.at[pl.ds(base, b_per_w)])

    return gather


_gather = _make_gather(HALF)


def _chain(full_x, round0, round1, books, cts, b2s):
    idx0 = round0(full_x, cts[0], b2s[0])
    g0 = _gather(books[0], jnp.reshape(idx0, (HALF,)))
    idx1, x1 = round1(full_x, g0, cts[1], b2s[1])
    g1 = _gather(books[1], jnp.reshape(idx1, (HALF,)))
    idx2, x2 = _round_h(x1, g1, cts[2], b2s[2])
    g2 = _gather(books[2], jnp.reshape(idx2, (HALF,)))
    idx3, x3 = _round_h(x2, g2, cts[3], b2s[3])
    g3 = _gather(books[3], jnp.reshape(idx3, (HALF,)))
    res = _sub_h(x3, g3)
    out = jnp.stack([jnp.reshape(i, (HALF,)) for i in (idx0, idx1, idx2, idx3)],
                    axis=-1)
    return out, res


def kernel(input, codebook_0, codebook_1, codebook_2, codebook_3):
    books = [codebook_0, codebook_1, codebook_2, codebook_3]
    cts = [jnp.transpose(c).astype(jnp.bfloat16) for c in books]
    b2s = [
        jnp.broadcast_to(jnp.sum(c * c, axis=-1)[None, :], (8, K)) for c in books
    ]

    out_a, res_a = _chain(input, _round0_a, _round1_a, books, cts, b2s)
    out_b, res_b = _chain(input, _round0_b, _round1_b, books, cts, b2s)

    output = jnp.concatenate([out_a, out_b], axis=0)
    res = jnp.concatenate([res_a, res_b], axis=0)
    return (output, res)


# speedup vs baseline: 1.0506x; 1.0183x over previous
import functools

import jax
import jax.numpy as jnp
from jax import lax
from jax.experimental import pallas as pl
from jax.experimental.pallas import tpu as pltpu
from jax.experimental.pallas import tpu_sc as plsc

N = 16384
K = 1024
DIM = 256
BN = 2048
HALF = N // 2
HGRID = HALF // BN


def _dist_argmin(x, ct_ref, b2_ref):
    t = x * x
    u = t[:, 0:128] + t[:, 128:256]
    ut = jnp.transpose(u)
    acc = ut[0:8, :]
    for v in range(1, 16):
        acc = acc + ut[8 * v:8 * v + 8, :]
    h4 = acc[0:4, :] + acc[4:8, :]
    h2 = h4[0:2, :] + h4[2:4, :]
    a2 = jnp.transpose(h2[0:1, :] + h2[1:2, :])
    s = lax.dot_general(
        x.astype(jnp.bfloat16), ct_ref[...],
        dimension_numbers=(((1,), (0,)), ((), ())),
        preferred_element_type=jnp.float32,
    )
    b2 = b2_ref[0:1, :]
    d2 = (a2 + b2) - 2.0 * s
    dc = jnp.clip(d2, 0.0, None)
    dist = dc * lax.rsqrt(dc)
    m = dist[:, 0:128]
    cid = jnp.zeros((BN, 128), jnp.int32)
    for j in range(1, K // 128):
        c = dist[:, j * 128:(j + 1) * 128]
        lt = c < m
        m = jnp.where(lt, c, m)
        cid = jnp.where(lt, j, cid)
    mm = jnp.min(m, axis=1, keepdims=True)
    lane = lax.broadcasted_iota(jnp.int32, (BN, 128), 1)
    cand = jnp.where(m == mm, cid * 128 + lane, jnp.int32(K))
    return jnp.reshape(jnp.min(cand, axis=1, keepdims=True), (BN // 128, 128))


def _round0_body(x_ref, ct_ref, b2_ref, idx_ref):
    idx_ref[...] = _dist_argmin(x_ref[...], ct_ref, b2_ref)


def _round_body(x_ref, g_ref, ct_ref, b2_ref, idx_ref, xout_ref):
    x = x_ref[...] - g_ref[...]
    xout_ref[...] = x
    idx_ref[...] = _dist_argmin(x, ct_ref, b2_ref)


def _sub_body(x_ref, g_ref, out_ref):
    out_ref[...] = x_ref[...] - g_ref[...]


_CTSPEC = pl.BlockSpec((DIM, K), lambda i: (0, 0))
_B2SPEC = pl.BlockSpec((8, K), lambda i: (0, 0))
_IDXSPEC = pl.BlockSpec((BN // 128, 128), lambda i: (i, 0))
_HSPEC = pl.BlockSpec((BN, DIM), lambda i: (i, 0))

_IDX_SHAPE = jax.ShapeDtypeStruct((HALF // 128, 128), jnp.int32)
_X_SHAPE = jax.ShapeDtypeStruct((HALF, DIM), jnp.float32)


def _xfull_spec(off):
    return pl.BlockSpec((BN, DIM), lambda i, o=off: (i + o, 0))


def _make_round0(off):
    return pl.pallas_call(
        _round0_body,
        grid=(HGRID,),
        in_specs=[_xfull_spec(off), _CTSPEC, _B2SPEC],
        out_specs=_IDXSPEC,
        out_shape=_IDX_SHAPE,
    )


def _make_round1(off):
    return pl.pallas_call(
        _round_body,
        grid=(HGRID,),
        in_specs=[_xfull_spec(off), _HSPEC, _CTSPEC, _B2SPEC],
        out_specs=[_IDXSPEC, _HSPEC],
        out_shape=[_IDX_SHAPE, _X_SHAPE],
    )


_round0_a = _make_round0(0)
_round0_b = _make_round0(HGRID)
_round1_a = _make_round1(0)
_round1_b = _make_round1(HGRID)

_round_h = pl.pallas_call(
    _round_body,
    grid=(HGRID,),
    in_specs=[_HSPEC, _HSPEC, _CTSPEC, _B2SPEC],
    out_specs=[_IDXSPEC, _HSPEC],
    out_shape=[_IDX_SHAPE, _X_SHAPE],
)

_sub_h = pl.pallas_call(
    _sub_body,
    grid=(HGRID,),
    in_specs=[_HSPEC, _HSPEC],
    out_specs=_HSPEC,
    out_shape=_X_SHAPE,
)


def _make_gather(rows):
    info = plsc.get_sparse_core_info()
    nc, ns = info.num_cores, info.num_subcores
    nw = nc * ns
    b_per_w = rows // nw
    mesh = plsc.VectorSubcoreMesh(core_axis_name="c", subcore_axis_name="s")

    @functools.partial(
        pl.kernel,
        out_type=jax.ShapeDtypeStruct((rows, DIM), jnp.float32),
        mesh=mesh,
        scratch_types=[
            pltpu.VMEM((b_per_w,), jnp.int32),
            pltpu.VMEM((b_per_w, DIM), jnp.float32),
            pltpu.SemaphoreType.DMA,
        ],
    )
    def gather(table_hbm, idx_hbm, out_hbm, idx_v, rows_v, sem):
        wid = lax.axis_index("s") * nc + lax.axis_index("c")
        base = wid * b_per_w
        pltpu.sync_copy(idx_hbm.at[pl.ds(base, b_per_w)], idx_v)
        half = b_per_w // 2
        g0 = pltpu.async_copy(table_hbm.at[idx_v.at[pl.ds(0, half)]],
                              rows_v.at[pl.ds(0, half)], sem)
        g1 = pltpu.async_copy(table_hbm.at[idx_v.at[pl.ds(half, half)]],
                              rows_v.at[pl.ds(half, half)], sem)
        g0.wait()
        g1.wait()
        pltpu.sync_copy(rows_v, out_hbm.at[pl.ds(base, b_per_w)])

    return gather


_gather = _make_gather(HALF)


def kernel(input, codebook_0, codebook_1, codebook_2, codebook_3):
    books = [codebook_0, codebook_1, codebook_2, codebook_3]
    cts = [jnp.transpose(c).astype(jnp.bfloat16) for c in books]
    b2s = [
        jnp.broadcast_to(jnp.sum(c * c, axis=-1)[None, :], (8, K)) for c in books
    ]

    ia0 = _round0_a(input, cts[0], b2s[0])
    ib0 = _round0_b(input, cts[0], b2s[0])
    ga0 = _gather(books[0], jnp.reshape(ia0, (HALF,)))
    gb0 = _gather(books[0], jnp.reshape(ib0, (HALF,)))
    ia1, xa1 = _round1_a(input, ga0, cts[1], b2s[1])
    ib1, xb1 = _round1_b(input, gb0, cts[1], b2s[1])
    ga1 = _gather(books[1], jnp.reshape(ia1, (HALF,)))
    gb1 = _gather(books[1], jnp.reshape(ib1, (HALF,)))
    ia2, xa2 = _round_h(xa1, ga1, cts[2], b2s[2])
    ib2, xb2 = _round_h(xb1, gb1, cts[2], b2s[2])
    ga2 = _gather(books[2], jnp.reshape(ia2, (HALF,)))
    gb2 = _gather(books[2], jnp.reshape(ib2, (HALF,)))
    ia3, xa3 = _round_h(xa2, ga2, cts[3], b2s[3])
    ib3, xb3 = _round_h(xb2, gb2, cts[3], b2s[3])
    ga3 = _gather(books[3], jnp.reshape(ia3, (HALF,)))
    gb3 = _gather(books[3], jnp.reshape(ib3, (HALF,)))
    res_a = _sub_h(xa3, ga3)
    res_b = _sub_h(xb3, gb3)

    out_a = jnp.stack([jnp.reshape(i, (HALF,)) for i in (ia0, ia1, ia2, ia3)],
                      axis=-1)
    out_b = jnp.stack([jnp.reshape(i, (HALF,)) for i in (ib0, ib1, ib2, ib3)],
                      axis=-1)
    output = jnp.concatenate([out_a, out_b], axis=0)
    res = jnp.concatenate([res_a, res_b], axis=0)
    return (output, res)
